# baseline (speedup 1.0000x reference)
import functools

import jax
import jax.numpy as jnp
import numpy as np
from jax.experimental import pallas as pl

N = 10000
E = 320000
D = 128
UNITS = 128
HEADS = 4
U = UNITS // HEADS
DEPTH = 3
T = 3

_NB = 1000


def _layer_tc_body(h_ref, agg0_ref, agg1_ref, wl_ref, bl_ref, a12_ref, c12_ref,
                   wm_ref, bm_ref, gb_ref, hk_ref, s12_ref, upd_ref, h_out_ref):
    h = h_ref[...]
    if agg0_ref is not None:
        h = h + jnp.maximum(agg0_ref[...] + agg1_ref[...], 0.0)
    h_out_ref[...] = h
    hk = jnp.dot(h, wl_ref[...], preferred_element_type=jnp.float32) + bl_ref[...]
    hk_ref[...] = hk
    s12_ref[...] = jnp.dot(hk, a12_ref[...], preferred_element_type=jnp.float32) + c12_ref[...]
    upd = jnp.maximum(jnp.dot(h, wm_ref[...], preferred_element_type=jnp.float32) + bm_ref[...], 0.0)
    upd_ref[...] = upd * gb_ref[0:1, :] + gb_ref[1:2, :]


def _layer_tc(h_or_upd, agg, wl, bl, a12, c12, wm, bm, gb):
    first = agg is None

    def body_first(h_ref, wl_ref, bl_ref, a12_ref, c12_ref, wm_ref, bm_ref, gb_ref,
                   hk_ref, s12_ref, upd_ref, h_out_ref):
        _layer_tc_body(h_ref, None, None, wl_ref, bl_ref, a12_ref, c12_ref,
                       wm_ref, bm_ref, gb_ref, hk_ref, s12_ref, upd_ref, h_out_ref)

    row = lambda i: (i, 0)
    full = lambda i: (0, 0)
    in_specs = [pl.BlockSpec((_NB, UNITS), row)]
    args = [h_or_upd]
    if not first:
        in_specs += [pl.BlockSpec((_NB, UNITS), row), pl.BlockSpec((_NB, UNITS), row)]
        args += [agg[0], agg[1]]
    in_specs += [
        pl.BlockSpec((UNITS, UNITS), full),
        pl.BlockSpec((1, UNITS), full),
        pl.BlockSpec((UNITS, 2 * HEADS), full),
        pl.BlockSpec((1, 2 * HEADS), full),
        pl.BlockSpec((UNITS, UNITS), full),
        pl.BlockSpec((1, UNITS), full),
        pl.BlockSpec((2, UNITS), full),
    ]
    args += [wl, bl, a12, c12, wm, bm, gb]
    out_specs = [
        pl.BlockSpec((_NB, UNITS), row),
        pl.BlockSpec((_NB, 2 * HEADS), row),
        pl.BlockSpec((_NB, UNITS), row),
        pl.BlockSpec((_NB, UNITS), row),
    ]
    out_shape = [
        jax.ShapeDtypeStruct((N, UNITS), jnp.float32),
        jax.ShapeDtypeStruct((N, 2 * HEADS), jnp.float32),
        jax.ShapeDtypeStruct((N, UNITS), jnp.float32),
        jax.ShapeDtypeStruct((N, UNITS), jnp.float32),
    ]
    fn = body_first if first else _layer_tc_body
    return pl.pallas_call(
        fn, grid=(N // _NB,), in_specs=in_specs, out_specs=out_specs,
        out_shape=out_shape)(*args)


def _set2set_body(upd_ref, agg0_ref, agg1_ref, wi_ref, wh_ref, bl_ref,
                  wp1_ref, bp1_ref, wp2_ref, bp2_ref, out_ref):
    h = upd_ref[...] + jnp.maximum(agg0_ref[...] + agg1_ref[...], 0.0)
    C = UNITS
    q_star = jnp.zeros((1, 2 * C), jnp.float32)
    cs = jnp.zeros((1, C), jnp.float32)
    hs = jnp.zeros((1, C), jnp.float32)
    for _ in range(T):
        z = (jnp.dot(q_star, wi_ref[...], preferred_element_type=jnp.float32)
             + jnp.dot(hs, wh_ref[...], preferred_element_type=jnp.float32)
             + bl_ref[...])
        i = z[:, 0:C]
        f = z[:, C:2 * C]
        g = z[:, 2 * C:3 * C]
        o = z[:, 3 * C:4 * C]
        cs = jax.nn.sigmoid(f) * cs + jax.nn.sigmoid(i) * jnp.tanh(g)
        hs = jax.nn.sigmoid(o) * jnp.tanh(cs)
        q = hs
        e = jnp.sum(h * q, axis=1, keepdims=True)
        m = jnp.max(e)
        ea = jnp.exp(e - m)
        alpha = ea / jnp.sum(ea)
        r = jnp.sum(alpha * h, axis=0, keepdims=True)
        q_star = jnp.concatenate([q, r], axis=1)
    p = jnp.maximum(jnp.dot(q_star, wp1_ref[...], preferred_element_type=jnp.float32) + bp1_ref[...], 0.0)
    p = jnp.maximum(jnp.dot(p, wp2_ref[...], preferred_element_type=jnp.float32) + bp2_ref[...], 0.0)
    out_ref[...] = p


def _set2set_tc(upd, agg, Wi, Wh, b_lstm, Wp1, bp1, Wp2, bp2):
    return pl.pallas_call(
        _set2set_body,
        out_shape=jax.ShapeDtypeStruct((1, UNITS // 2), jnp.float32),
    )(upd, agg[0], agg[1], Wi, Wh, b_lstm.reshape(1, -1),
      Wp1, bp1.reshape(1, -1), Wp2, bp2.reshape(1, -1))


def _edge_pass(s12, M, hk, src, dst):
    z = s12[src, :HEADS] + s12[dst, HEADS:]
    logit = jnp.maximum(z, 0.2 * z)
    ex = jnp.exp(logit - M)
    den = jax.ops.segment_sum(ex, dst, num_segments=N)
    alpha = ex / (den[dst] + 1e-9)
    w = jnp.repeat(alpha, U, axis=1)
    agg = jax.ops.segment_sum(w * hk[src], dst, num_segments=N)
    return jnp.stack([agg, jnp.zeros_like(agg)], axis=0)


def kernel(node_attributes, edge_indices, W_att, b_att, a_att, W_mlp, b_mlp,
           gamma, beta, Wi, Wh, b_lstm, Wp1, bp1, Wp2, bp2):
    x = node_attributes
    dst = edge_indices[:, 0].astype(jnp.int32)
    src = edge_indices[:, 1].astype(jnp.int32)

    a1 = np.zeros((HEADS * U, HEADS), np.float32)
    a2 = np.zeros((HEADS * U, HEADS), np.float32)
    mask1 = np.zeros((HEADS, U, HEADS), np.float32)
    mask2 = np.zeros((HEADS, U, HEADS), np.float32)
    for hd in range(HEADS):
        mask1[hd, :, hd] = 1.0
        mask2[hd, :, hd] = 1.0
    mask1 = jnp.asarray(mask1.reshape(HEADS * U, HEADS))
    mask2 = jnp.asarray(mask2.reshape(HEADS * U, HEADS))

    agg = None
    h = x
    upd = None
    for l in range(DEPTH):
        wl = jnp.transpose(W_att[l], (1, 0, 2)).reshape(D, HEADS * U)
        bl = b_att[l].reshape(1, HEADS * U)
        A1 = mask1 * a_att[l, :, :U].reshape(HEADS * U, 1)
        A2 = mask2 * a_att[l, :, U:2 * U].reshape(HEADS * U, 1)
        a12 = jnp.concatenate([A1, A2], axis=1)
        c12 = jnp.concatenate([jnp.zeros((HEADS,), jnp.float32), a_att[l, :, 2 * U]]).reshape(1, 2 * HEADS)
        gb = jnp.stack([gamma[l], beta[l]], axis=0)
        hk, s12, upd, h = _layer_tc(h if l == 0 else upd, agg, wl, bl, a12, c12,
                                    W_mlp[l], b_mlp[l].reshape(1, UNITS), gb)
        Mh = jnp.max(s12[:, :HEADS], axis=0) + jnp.max(s12[:, HEADS:], axis=0)
        M = jnp.maximum(Mh, 0.2 * Mh)
        agg = _edge_pass(s12, M, hk, src, dst)

    p = _set2set_tc(upd, agg, Wi, Wh, b_lstm, Wp1, bp1, Wp2, bp2)
    return (p, jnp.stack([p, p], axis=1))


# speedup vs baseline: 18.8858x; 18.8858x over previous
import functools

import jax
import jax.numpy as jnp
import numpy as np
from jax import lax
from jax.experimental import pallas as pl
from jax.experimental.pallas import tpu as pltpu
from jax.experimental.pallas import tpu_sc as plsc

N = 10000
E = 320000
D = 128
UNITS = 128
HEADS = 4
U = UNITS // HEADS
DEPTH = 3
T = 3

_NC = 2
_NS = 16
_L = 16
_NW = _NC * _NS
_EW = E // _NW
_NP = 10240
_BLK = 80
_DW = 16
_NBLK = _EW // _BLK


def _sc_mesh(nc=_NC):
    return plsc.VectorSubcoreMesh(core_axis_name="c", subcore_axis_name="s",
                                  num_cores=nc, num_subcores=_NS)

_NB = 1000


def _layer_tc_body(h_ref, agg_ref, wl_ref, bl_ref, a12_ref, c12_ref,
                   wm_ref, bm_ref, gb_ref, hk_ref, s12_ref, upd_ref, h_out_ref):
    h = h_ref[...]
    if agg_ref is not None:
        h = h + jnp.maximum(agg_ref[...], 0.0)
    h_out_ref[...] = h
    hk = jnp.dot(h, wl_ref[...], preferred_element_type=jnp.float32) + bl_ref[...]
    hk_ref[...] = hk
    s12_ref[...] = jnp.dot(hk, a12_ref[...], preferred_element_type=jnp.float32) + c12_ref[...]
    upd = jnp.maximum(jnp.dot(h, wm_ref[...], preferred_element_type=jnp.float32) + bm_ref[...], 0.0)
    upd_ref[...] = upd * gb_ref[0:1, :] + gb_ref[1:2, :]


def _layer_tc(h_or_upd, agg, wl, bl, a12, c12, wm, bm, gb):
    first = agg is None

    def body_first(h_ref, wl_ref, bl_ref, a12_ref, c12_ref, wm_ref, bm_ref, gb_ref,
                   hk_ref, s12_ref, upd_ref, h_out_ref):
        _layer_tc_body(h_ref, None, wl_ref, bl_ref, a12_ref, c12_ref,
                       wm_ref, bm_ref, gb_ref, hk_ref, s12_ref, upd_ref, h_out_ref)

    row = lambda i: (i, 0)
    full = lambda i: (0, 0)
    in_specs = [pl.BlockSpec((_NB, UNITS), row)]
    args = [h_or_upd]
    if not first:
        in_specs += [pl.BlockSpec((_NB, UNITS), row)]
        args += [agg]
    in_specs += [
        pl.BlockSpec((UNITS, UNITS), full),
        pl.BlockSpec((1, UNITS), full),
        pl.BlockSpec((UNITS, 2 * HEADS), full),
        pl.BlockSpec((1, 2 * HEADS), full),
        pl.BlockSpec((UNITS, UNITS), full),
        pl.BlockSpec((1, UNITS), full),
        pl.BlockSpec((2, UNITS), full),
    ]
    args += [wl, bl, a12, c12, wm, bm, gb]
    out_specs = [
        pl.BlockSpec((_NB, UNITS), row),
        pl.BlockSpec((_NB, 2 * HEADS), row),
        pl.BlockSpec((_NB, UNITS), row),
        pl.BlockSpec((_NB, UNITS), row),
    ]
    out_shape = [
        jax.ShapeDtypeStruct((N, UNITS), jnp.float32),
        jax.ShapeDtypeStruct((N, 2 * HEADS), jnp.float32),
        jax.ShapeDtypeStruct((N, UNITS), jnp.float32),
        jax.ShapeDtypeStruct((N, UNITS), jnp.float32),
    ]
    fn = body_first if first else _layer_tc_body
    return pl.pallas_call(
        fn, grid=(N // _NB,), in_specs=in_specs, out_specs=out_specs,
        out_shape=out_shape)(*args)


def _set2set_body(upd_ref, agg_ref, wi_ref, wh_ref, bl_ref,
                  wp1_ref, bp1_ref, wp2_ref, bp2_ref, out_ref):
    h = upd_ref[...] + jnp.maximum(agg_ref[...], 0.0)
    C = UNITS
    q_star = jnp.zeros((1, 2 * C), jnp.float32)
    cs = jnp.zeros((1, C), jnp.float32)
    hs = jnp.zeros((1, C), jnp.float32)
    for _ in range(T):
        z = (jnp.dot(q_star, wi_ref[...], preferred_element_type=jnp.float32)
             + jnp.dot(hs, wh_ref[...], preferred_element_type=jnp.float32)
             + bl_ref[...])
        i = z[:, 0:C]
        f = z[:, C:2 * C]
        g = z[:, 2 * C:3 * C]
        o = z[:, 3 * C:4 * C]
        cs = jax.nn.sigmoid(f) * cs + jax.nn.sigmoid(i) * jnp.tanh(g)
        hs = jax.nn.sigmoid(o) * jnp.tanh(cs)
        q = hs
        e = jnp.sum(h * q, axis=1, keepdims=True)
        m = jnp.max(e)
        ea = jnp.exp(e - m)
        alpha = ea / jnp.sum(ea)
        r = jnp.sum(alpha * h, axis=0, keepdims=True)
        q_star = jnp.concatenate([q, r], axis=1)
    p = jnp.maximum(jnp.dot(q_star, wp1_ref[...], preferred_element_type=jnp.float32) + bp1_ref[...], 0.0)
    p = jnp.maximum(jnp.dot(p, wp2_ref[...], preferred_element_type=jnp.float32) + bp2_ref[...], 0.0)
    out_ref[...] = p


def _set2set_tc(upd, agg, Wi, Wh, b_lstm, Wp1, bp1, Wp2, bp2):
    C = UNITS
    zero2 = lambda i: (0, 0)
    in_specs = [
        pl.BlockSpec((N, C), zero2),
        pl.BlockSpec((N, C), zero2),
        pl.BlockSpec((2 * C, 4 * C), zero2),
        pl.BlockSpec((C, 4 * C), zero2),
        pl.BlockSpec((1, 4 * C), zero2),
        pl.BlockSpec((2 * C, C), zero2),
        pl.BlockSpec((1, C), zero2),
        pl.BlockSpec((C, C // 2), zero2),
        pl.BlockSpec((1, C // 2), zero2),
    ]
    return pl.pallas_call(
        _set2set_body,
        grid=(1,),
        in_specs=in_specs,
        out_specs=pl.BlockSpec((1, C // 2), zero2),
        out_shape=jax.ShapeDtypeStruct((1, UNITS // 2), jnp.float32),
    )(upd, agg, Wi, Wh, b_lstm.reshape(1, -1),
      Wp1, bp1.reshape(1, -1), Wp2, bp2.reshape(1, -1))


def _sc_den(src, dst, s12flat, mrep, zden):
    @functools.partial(
        pl.kernel,
        out_type=(jax.ShapeDtypeStruct((_NC, _NP, _DW), jnp.float32),
                  jax.ShapeDtypeStruct((E * HEADS,), jnp.float32)),
        mesh=_sc_mesh(),
        compiler_params=pltpu.CompilerParams(needs_layout_passes=False, use_tc_tiling_on_sc=False),
        scratch_types=[
            pltpu.VMEM((N * 2 * HEADS,), jnp.float32),
            pltpu.VMEM((_BLK,), jnp.int32),
            pltpu.VMEM((_BLK,), jnp.int32),
            pltpu.VMEM((_BLK, _DW), jnp.float32),
            pltpu.VMEM((_BLK * HEADS,), jnp.float32),
            pltpu.VMEM((HEADS * _L,), jnp.float32),
            pltpu.VMEM_SHARED((_NP, _DW), jnp.float32),
        ],
    )
    def k(src_h, dst_h, s12_h, m_h, zden_h, den_out, ex_out,
          s12_v, srcb, dstb, exb, exbf, m_v, den_sh):
        c = lax.axis_index("c")
        s = lax.axis_index("s")
        base = c * (E // _NC) + s * _EW
        rows = _NP // _NS
        pltpu.sync_copy(zden_h.at[pl.ds(s * rows, rows)],
                        den_sh.at[pl.ds(s * rows, rows)])
        pltpu.sync_copy(s12_h, s12_v)
        pltpu.sync_copy(m_h, m_v)
        zv = jnp.zeros((_L,), jnp.float32)

        def zexr(e, carry):
            exb[e, pl.ds(0, _L)] = zv
            return carry

        lax.fori_loop(0, _BLK, zexr, 0)
        plsc.subcore_barrier()
        iota = lax.iota(jnp.int32, _L)

        def blk(j, carry):
            off = base + j * _BLK
            pltpu.sync_copy(src_h.at[pl.ds(off, _BLK)], srcb)
            pltpu.sync_copy(dst_h.at[pl.ds(off, _BLK)], dstb)

            def grp(i, carry2):
                srcv = srcb[pl.ds(i * _L, _L)]
                dstv = dstb[pl.ds(i * _L, _L)]
                si = srcv * (2 * HEADS)
                di = dstv * (2 * HEADS) + HEADS
                row = i * _L + iota
                for hd in range(HEADS):
                    s1 = plsc.load_gather(s12_v, [si + hd])
                    s2 = plsc.load_gather(s12_v, [di + hd])
                    z = s1 + s2
                    lg = jnp.maximum(z, 0.2 * z)
                    exv = jnp.exp(lg - m_v[pl.ds(hd * _L, _L)])
                    plsc.store_scatter(exb, [row, jnp.full((_L,), hd, jnp.int32)], exv)
                    plsc.store_scatter(exbf, [row * HEADS + hd], exv)
                return carry2

            lax.fori_loop(0, _BLK // _L, grp, 0)
            pltpu.sync_copy(exbf, ex_out.at[pl.ds(off * HEADS, _BLK * HEADS)])
            pltpu.sync_copy(exb, den_sh.at[dstb], add=True)
            return carry

        lax.fori_loop(0, _NBLK, blk, 0)
        plsc.subcore_barrier()
        pltpu.sync_copy(den_sh.at[pl.ds(s * rows, rows)],
                        den_out.at[c, pl.ds(s * rows, rows)])

    return k(src, dst, s12flat, mrep, zden)


def _sc_alpha(dst, exbuf, den):
    @functools.partial(
        pl.kernel,
        out_type=jax.ShapeDtypeStruct((E * HEADS,), jnp.float32),
        mesh=_sc_mesh(),
        compiler_params=pltpu.CompilerParams(needs_layout_passes=False),
        scratch_types=[
            pltpu.VMEM((_BLK,), jnp.int32),
            pltpu.VMEM((_BLK * HEADS,), jnp.float32),
            pltpu.VMEM((_NP * HEADS,), jnp.float32),
            pltpu.VMEM((_BLK * HEADS,), jnp.float32),
            pltpu.SemaphoreType.DMA,
        ],
    )
    def k(dst_h, ex_h, den_h, al_out, dstb, exb, den_v, ab, sem):
        c = lax.axis_index("c")
        s = lax.axis_index("s")
        base = c * (E // _NC) + s * _EW
        iota = lax.iota(jnp.int32, _L)
        pltpu.sync_copy(den_h, den_v)

        def blk(j, carry):
            off = base + j * _BLK
            pltpu.sync_copy(dst_h.at[pl.ds(off, _BLK)], dstb)
            pltpu.sync_copy(ex_h.at[pl.ds(off * HEADS, _BLK * HEADS)], exb)

            def grp(i, carry2):
                dstv = dstb[pl.ds(i * _L, _L)]
                row = i * _L + iota
                for hd in range(HEADS):
                    exv = plsc.load_gather(exb, [row * HEADS + hd])
                    denv = plsc.load_gather(den_v, [dstv * HEADS + hd])
                    plsc.store_scatter(ab, [row * HEADS + hd], exv / (denv + 1e-9))
                return carry2

            lax.fori_loop(0, _BLK // _L, grp, 0)
            pltpu.sync_copy(ab, al_out.at[pl.ds(off * HEADS, _BLK * HEADS)])
            return carry

        lax.fori_loop(0, _NBLK, blk, 0)

    return k(dst, exbuf, den)


def _sc_agg(src, dst, alpha, hk):
    @functools.partial(
        pl.kernel,
        out_type=jax.ShapeDtypeStruct((_NP, UNITS), jnp.float32),
        mesh=_sc_mesh(1),
        compiler_params=pltpu.CompilerParams(needs_layout_passes=False),
        scratch_types=[
            pltpu.VMEM((_BLK,), jnp.int32),
            pltpu.VMEM((_BLK,), jnp.int32),
            pltpu.VMEM((_BLK * HEADS,), jnp.float32),
            pltpu.VMEM((_BLK, UNITS), jnp.float32),
            pltpu.VMEM_SHARED((_NP, UNITS), jnp.float32),
            pltpu.SemaphoreType.DMA,
        ],
    )
    def k(src_h, dst_h, al_h, hk_h, agg_out,
          srcb, dstb, ab, rows_v, agg_sh, sem):
        s = lax.axis_index("s")
        base = s * (E // _NS)
        rows = _NP // _NS
        nblk = (E // _NS) // _BLK
        zv = jnp.zeros((_L,), jnp.float32)

        def zrow(e, carry):
            for kk in range(UNITS // _L):
                rows_v[e, pl.ds(kk * _L, _L)] = zv
            return carry

        lax.fori_loop(0, _BLK, zrow, 0)
        for i in range(rows // _BLK):
            pltpu.sync_copy(rows_v, agg_sh.at[pl.ds(s * rows + i * _BLK, _BLK)])
        plsc.subcore_barrier()

        def blk(j, carry):
            off = base + j * _BLK
            pltpu.sync_copy(src_h.at[pl.ds(off, _BLK)], srcb)
            pltpu.sync_copy(dst_h.at[pl.ds(off, _BLK)], dstb)
            pltpu.sync_copy(al_h.at[pl.ds(off * HEADS, _BLK * HEADS)], ab)
            pltpu.async_copy(hk_h.at[srcb], rows_v, sem).wait()

            def edge(e, carry3):
                for hd in range(HEADS):
                    av = plsc.load_gather(ab, [jnp.full((_L,), e * HEADS + hd, jnp.int32)])
                    for kk in range(U // _L):
                        c0 = hd * U + kk * _L
                        rows_v[e, pl.ds(c0, _L)] = rows_v[e, pl.ds(c0, _L)] * av
                return carry3

            lax.fori_loop(0, _BLK, edge, 0)
            pltpu.sync_copy(rows_v, agg_sh.at[dstb], add=True)
            return carry

        lax.fori_loop(0, nblk, blk, 0)
        plsc.subcore_barrier()
        pltpu.sync_copy(agg_sh.at[pl.ds(s * rows, rows)],
                        agg_out.at[pl.ds(s * rows, rows)])

    return k(src, dst, alpha, hk)


def _edge_pass(s12, M, hk, src, dst, zden):
    mrep = jnp.broadcast_to(M[:, None], (HEADS, _L)).reshape(-1)
    den_parts, exbuf = _sc_den(src, dst, s12.reshape(-1), mrep, zden)
    den = den_parts[0] + den_parts[1]
    alpha = _sc_alpha(dst, exbuf, den[:, :HEADS].reshape(-1))
    return _sc_agg(src, dst, alpha, hk)


def kernel(node_attributes, edge_indices, W_att, b_att, a_att, W_mlp, b_mlp,
           gamma, beta, Wi, Wh, b_lstm, Wp1, bp1, Wp2, bp2):
    x = node_attributes
    dst = edge_indices[:, 0].astype(jnp.int32)
    src = edge_indices[:, 1].astype(jnp.int32)

    a1 = np.zeros((HEADS * U, HEADS), np.float32)
    a2 = np.zeros((HEADS * U, HEADS), np.float32)
    mask1 = np.zeros((HEADS, U, HEADS), np.float32)
    mask2 = np.zeros((HEADS, U, HEADS), np.float32)
    for hd in range(HEADS):
        mask1[hd, :, hd] = 1.0
        mask2[hd, :, hd] = 1.0
    mask1 = jnp.asarray(mask1.reshape(HEADS * U, HEADS))
    mask2 = jnp.asarray(mask2.reshape(HEADS * U, HEADS))

    zden = jnp.zeros((_NP, _DW), jnp.float32)
    agg = None
    h = x
    upd = None
    for l in range(DEPTH):
        wl = jnp.transpose(W_att[l], (1, 0, 2)).reshape(D, HEADS * U)
        bl = b_att[l].reshape(1, HEADS * U)
        A1 = mask1 * a_att[l, :, :U].reshape(HEADS * U, 1)
        A2 = mask2 * a_att[l, :, U:2 * U].reshape(HEADS * U, 1)
        a12 = jnp.concatenate([A1, A2], axis=1)
        c12 = jnp.concatenate([jnp.zeros((HEADS,), jnp.float32), a_att[l, :, 2 * U]]).reshape(1, 2 * HEADS)
        gb = jnp.stack([gamma[l], beta[l]], axis=0)
        hk, s12, upd, h = _layer_tc(h if l == 0 else upd, agg, wl, bl, a12, c12,
                                    W_mlp[l], b_mlp[l].reshape(1, UNITS), gb)
        Mh = jnp.max(s12[:, :HEADS], axis=0) + jnp.max(s12[:, HEADS:], axis=0)
        M = jnp.maximum(Mh, 0.2 * Mh)
        agg = _edge_pass(s12, M, hk, src, dst, zden)

    p = _set2set_tc(upd, agg, Wi, Wh, b_lstm, Wp1, bp1, Wp2, bp2)
    return (p, jnp.stack([p, p], axis=1))


# speedup vs baseline: 22.9277x; 1.2140x over previous
import functools

import jax
import jax.numpy as jnp
import numpy as np
from jax import lax
from jax.experimental import pallas as pl
from jax.experimental.pallas import tpu as pltpu
from jax.experimental.pallas import tpu_sc as plsc

N = 10000
E = 320000
D = 128
UNITS = 128
HEADS = 4
U = UNITS // HEADS
DEPTH = 3
T = 3

_NC = 2
_NS = 16
_L = 16
_NW = _NC * _NS
_EW = E // _NW
_NP = 10240
_BLK = 80
_DW = 16
_NBLK = _EW // _BLK


def _sc_mesh(nc=_NC):
    return plsc.VectorSubcoreMesh(core_axis_name="c", subcore_axis_name="s",
                                  num_cores=nc, num_subcores=_NS)

_NB = 1000


def _layer_tc_body(h_ref, agg0_ref, agg1_ref, wl_ref, bl_ref, a12_ref, c12_ref,
                   wm_ref, bm_ref, gb_ref, hk_ref, s12_ref, upd_ref, h_out_ref):
    h = h_ref[...]
    if agg0_ref is not None:
        att = jnp.concatenate([agg0_ref[...], agg1_ref[...]], axis=1)
        h = h + jnp.maximum(att, 0.0)
    h_out_ref[...] = h
    hk = jnp.dot(h, wl_ref[...], preferred_element_type=jnp.float32) + bl_ref[...]
    hk_ref[...] = hk
    s12_ref[...] = jnp.dot(hk, a12_ref[...], preferred_element_type=jnp.float32) + c12_ref[...]
    upd = jnp.maximum(jnp.dot(h, wm_ref[...], preferred_element_type=jnp.float32) + bm_ref[...], 0.0)
    upd_ref[...] = upd * gb_ref[0:1, :] + gb_ref[1:2, :]


def _layer_tc(h_or_upd, agg, wl, bl, a12, c12, wm, bm, gb):
    first = agg is None

    def body_first(h_ref, wl_ref, bl_ref, a12_ref, c12_ref, wm_ref, bm_ref, gb_ref,
                   hk_ref, s12_ref, upd_ref, h_out_ref):
        _layer_tc_body(h_ref, None, None, wl_ref, bl_ref, a12_ref, c12_ref,
                       wm_ref, bm_ref, gb_ref, hk_ref, s12_ref, upd_ref, h_out_ref)

    row = lambda i: (i, 0)
    full = lambda i: (0, 0)
    in_specs = [pl.BlockSpec((_NB, UNITS), row)]
    args = [h_or_upd]
    if not first:
        in_specs += [pl.BlockSpec((_NB, UNITS // 2), row), pl.BlockSpec((_NB, UNITS // 2), row)]
        args += [agg[0], agg[1]]
    in_specs += [
        pl.BlockSpec((UNITS, UNITS), full),
        pl.BlockSpec((1, UNITS), full),
        pl.BlockSpec((UNITS, 2 * HEADS), full),
        pl.BlockSpec((1, 2 * HEADS), full),
        pl.BlockSpec((UNITS, UNITS), full),
        pl.BlockSpec((1, UNITS), full),
        pl.BlockSpec((2, UNITS), full),
    ]
    args += [wl, bl, a12, c12, wm, bm, gb]
    out_specs = [
        pl.BlockSpec((_NB, UNITS), row),
        pl.BlockSpec((_NB, 2 * HEADS), row),
        pl.BlockSpec((_NB, UNITS), row),
        pl.BlockSpec((_NB, UNITS), row),
    ]
    out_shape = [
        jax.ShapeDtypeStruct((N, UNITS), jnp.float32),
        jax.ShapeDtypeStruct((N, 2 * HEADS), jnp.float32),
        jax.ShapeDtypeStruct((N, UNITS), jnp.float32),
        jax.ShapeDtypeStruct((N, UNITS), jnp.float32),
    ]
    fn = body_first if first else _layer_tc_body
    return pl.pallas_call(
        fn, grid=(N // _NB,), in_specs=in_specs, out_specs=out_specs,
        out_shape=out_shape)(*args)


def _set2set_body(upd_ref, agg0_ref, agg1_ref, wi_ref, wh_ref, bl_ref,
                  wp1_ref, bp1_ref, wp2_ref, bp2_ref, out_ref):
    att = jnp.concatenate([agg0_ref[...], agg1_ref[...]], axis=1)
    h = upd_ref[...] + jnp.maximum(att, 0.0)
    C = UNITS
    q_star = jnp.zeros((1, 2 * C), jnp.float32)
    cs = jnp.zeros((1, C), jnp.float32)
    hs = jnp.zeros((1, C), jnp.float32)
    for _ in range(T):
        z = (jnp.dot(q_star, wi_ref[...], preferred_element_type=jnp.float32)
             + jnp.dot(hs, wh_ref[...], preferred_element_type=jnp.float32)
             + bl_ref[...])
        i = z[:, 0:C]
        f = z[:, C:2 * C]
        g = z[:, 2 * C:3 * C]
        o = z[:, 3 * C:4 * C]
        cs = jax.nn.sigmoid(f) * cs + jax.nn.sigmoid(i) * jnp.tanh(g)
        hs = jax.nn.sigmoid(o) * jnp.tanh(cs)
        q = hs
        e = jnp.sum(h * q, axis=1, keepdims=True)
        m = jnp.max(e)
        ea = jnp.exp(e - m)
        alpha = ea / jnp.sum(ea)
        r = jnp.sum(alpha * h, axis=0, keepdims=True)
        q_star = jnp.concatenate([q, r], axis=1)
    p = jnp.maximum(jnp.dot(q_star, wp1_ref[...], preferred_element_type=jnp.float32) + bp1_ref[...], 0.0)
    p = jnp.maximum(jnp.dot(p, wp2_ref[...], preferred_element_type=jnp.float32) + bp2_ref[...], 0.0)
    out_ref[...] = p


def _set2set_tc(upd, agg, Wi, Wh, b_lstm, Wp1, bp1, Wp2, bp2):
    C = UNITS
    zero2 = lambda i: (0, 0)
    in_specs = [
        pl.BlockSpec((N, C), zero2),
        pl.BlockSpec((N, C // 2), zero2),
        pl.BlockSpec((N, C // 2), zero2),
        pl.BlockSpec((2 * C, 4 * C), zero2),
        pl.BlockSpec((C, 4 * C), zero2),
        pl.BlockSpec((1, 4 * C), zero2),
        pl.BlockSpec((2 * C, C), zero2),
        pl.BlockSpec((1, C), zero2),
        pl.BlockSpec((C, C // 2), zero2),
        pl.BlockSpec((1, C // 2), zero2),
    ]
    return pl.pallas_call(
        _set2set_body,
        grid=(1,),
        in_specs=in_specs,
        out_specs=pl.BlockSpec((1, C // 2), zero2),
        out_shape=jax.ShapeDtypeStruct((1, UNITS // 2), jnp.float32),
    )(upd, agg[0], agg[1], Wi, Wh, b_lstm.reshape(1, -1),
      Wp1, bp1.reshape(1, -1), Wp2, bp2.reshape(1, -1))


def _sc_den(src, dst, s12flat, mrep, zden):
    @functools.partial(
        pl.kernel,
        out_type=(jax.ShapeDtypeStruct((_NC, _NP, _DW), jnp.float32),
                  jax.ShapeDtypeStruct((E * HEADS,), jnp.float32)),
        mesh=_sc_mesh(),
        compiler_params=pltpu.CompilerParams(needs_layout_passes=False, use_tc_tiling_on_sc=False),
        scratch_types=[
            pltpu.VMEM((N * 2 * HEADS,), jnp.float32),
            pltpu.VMEM((_BLK,), jnp.int32),
            pltpu.VMEM((_BLK,), jnp.int32),
            pltpu.VMEM((_BLK, _DW), jnp.float32),
            pltpu.VMEM((_BLK * HEADS,), jnp.float32),
            pltpu.VMEM((HEADS * _L,), jnp.float32),
            pltpu.VMEM_SHARED((_NP, _DW), jnp.float32),
        ],
    )
    def k(src_h, dst_h, s12_h, m_h, zden_h, den_out, ex_out,
          s12_v, srcb, dstb, exb, exbf, m_v, den_sh):
        c = lax.axis_index("c")
        s = lax.axis_index("s")
        base = c * (E // _NC) + s * _EW
        rows = _NP // _NS
        pltpu.sync_copy(zden_h.at[pl.ds(s * rows, rows)],
                        den_sh.at[pl.ds(s * rows, rows)])
        pltpu.sync_copy(s12_h, s12_v)
        pltpu.sync_copy(m_h, m_v)
        zv = jnp.zeros((_L,), jnp.float32)

        def zexr(e, carry):
            exb[e, pl.ds(0, _L)] = zv
            return carry

        lax.fori_loop(0, _BLK, zexr, 0)
        plsc.subcore_barrier()
        iota = lax.iota(jnp.int32, _L)

        def blk(j, carry):
            off = base + j * _BLK
            pltpu.sync_copy(src_h.at[pl.ds(off, _BLK)], srcb)
            pltpu.sync_copy(dst_h.at[pl.ds(off, _BLK)], dstb)

            def grp(i, carry2):
                srcv = srcb[pl.ds(i * _L, _L)]
                dstv = dstb[pl.ds(i * _L, _L)]
                si = srcv * (2 * HEADS)
                di = dstv * (2 * HEADS) + HEADS
                row = i * _L + iota
                for hd in range(HEADS):
                    s1 = plsc.load_gather(s12_v, [si + hd])
                    s2 = plsc.load_gather(s12_v, [di + hd])
                    z = s1 + s2
                    lg = jnp.maximum(z, 0.2 * z)
                    exv = jnp.exp(lg - m_v[pl.ds(hd * _L, _L)])
                    plsc.store_scatter(exb, [row, jnp.full((_L,), hd, jnp.int32)], exv)
                    plsc.store_scatter(exbf, [row * HEADS + hd], exv)
                return carry2

            lax.fori_loop(0, _BLK // _L, grp, 0)
            pltpu.sync_copy(exbf, ex_out.at[pl.ds(off * HEADS, _BLK * HEADS)])
            pltpu.sync_copy(exb, den_sh.at[dstb], add=True)
            return carry

        lax.fori_loop(0, _NBLK, blk, 0)
        plsc.subcore_barrier()
        pltpu.sync_copy(den_sh.at[pl.ds(s * rows, rows)],
                        den_out.at[c, pl.ds(s * rows, rows)])

    return k(src, dst, s12flat, mrep, zden)


def _sc_alpha(dst, exbuf, den):
    @functools.partial(
        pl.kernel,
        out_type=jax.ShapeDtypeStruct((E * HEADS,), jnp.float32),
        mesh=_sc_mesh(),
        compiler_params=pltpu.CompilerParams(needs_layout_passes=False),
        scratch_types=[
            pltpu.VMEM((_BLK,), jnp.int32),
            pltpu.VMEM((_BLK * HEADS,), jnp.float32),
            pltpu.VMEM((_NP * HEADS,), jnp.float32),
            pltpu.VMEM((_BLK * HEADS,), jnp.float32),
            pltpu.SemaphoreType.DMA,
        ],
    )
    def k(dst_h, ex_h, den_h, al_out, dstb, exb, den_v, ab, sem):
        c = lax.axis_index("c")
        s = lax.axis_index("s")
        base = c * (E // _NC) + s * _EW
        iota = lax.iota(jnp.int32, _L)
        pltpu.sync_copy(den_h, den_v)

        def blk(j, carry):
            off = base + j * _BLK
            pltpu.sync_copy(dst_h.at[pl.ds(off, _BLK)], dstb)
            pltpu.sync_copy(ex_h.at[pl.ds(off * HEADS, _BLK * HEADS)], exb)

            def grp(i, carry2):
                dstv = dstb[pl.ds(i * _L, _L)]
                row = i * _L + iota
                for hd in range(HEADS):
                    exv = plsc.load_gather(exb, [row * HEADS + hd])
                    denv = plsc.load_gather(den_v, [dstv * HEADS + hd])
                    plsc.store_scatter(ab, [row * HEADS + hd], exv / (denv + 1e-9))
                return carry2

            lax.fori_loop(0, _BLK // _L, grp, 0)
            pltpu.sync_copy(ab, al_out.at[pl.ds(off * HEADS, _BLK * HEADS)])
            return carry

        lax.fori_loop(0, _NBLK, blk, 0)

    return k(dst, exbuf, den)


def _sc_agg(src, dst, alpha, hk2):
    HALF = UNITS // 2

    @functools.partial(
        pl.kernel,
        out_type=jax.ShapeDtypeStruct((_NC, _NP, HALF), jnp.float32),
        mesh=_sc_mesh(),
        compiler_params=pltpu.CompilerParams(needs_layout_passes=False, use_tc_tiling_on_sc=False),
        scratch_types=[
            pltpu.VMEM((_BLK,), jnp.int32),
            pltpu.VMEM((_BLK,), jnp.int32),
            pltpu.VMEM((_BLK * HEADS,), jnp.float32),
            pltpu.VMEM((_BLK, HALF), jnp.float32),
            pltpu.VMEM_SHARED((_NP, HALF), jnp.float32),
            pltpu.SemaphoreType.DMA,
        ],
    )
    def k(src_h, dst_h, al_h, hk_h, agg_out,
          srcb, dstb, ab, rows_v, agg_sh, sem):
        c = lax.axis_index("c")
        s = lax.axis_index("s")
        base = s * (E // _NS)
        rows = _NP // _NS
        nblk = (E // _NS) // _BLK
        zv = jnp.zeros((_L,), jnp.float32)
        roff = c * N

        def zrow(e, carry):
            for kk in range(HALF // _L):
                rows_v[e, pl.ds(kk * _L, _L)] = zv
            return carry

        lax.fori_loop(0, _BLK, zrow, 0)
        for i in range(rows // _BLK):
            pltpu.sync_copy(rows_v, agg_sh.at[pl.ds(s * rows + i * _BLK, _BLK)])
        plsc.subcore_barrier()

        def blk(j, carry):
            off = base + j * _BLK
            pltpu.sync_copy(src_h.at[pl.ds(off, _BLK)], srcb)
            pltpu.sync_copy(dst_h.at[pl.ds(off, _BLK)], dstb)
            pltpu.sync_copy(al_h.at[pl.ds(off * HEADS, _BLK * HEADS)], ab)

            def shft(i, carry0):
                sl = pl.ds(i * _L, _L)
                srcb[sl] = srcb[sl] + roff
                return carry0

            lax.fori_loop(0, _BLK // _L, shft, 0)
            pltpu.async_copy(hk_h.at[srcb], rows_v, sem).wait()

            def edge(e, carry3):
                for hd in range(HEADS // _NC):
                    av = plsc.load_gather(
                        ab, [jnp.full((_L,), e * HEADS + hd, jnp.int32) + 2 * c])
                    for kk in range(U // _L):
                        c0 = hd * U + kk * _L
                        rows_v[e, pl.ds(c0, _L)] = rows_v[e, pl.ds(c0, _L)] * av
                return carry3

            lax.fori_loop(0, _BLK, edge, 0)
            pltpu.sync_copy(rows_v, agg_sh.at[dstb], add=True)
            return carry

        lax.fori_loop(0, nblk, blk, 0)
        plsc.subcore_barrier()
        pltpu.sync_copy(agg_sh.at[pl.ds(s * rows, rows)],
                        agg_out.at[c, pl.ds(s * rows, rows)])

    return k(src, dst, alpha, hk2)


def _edge_pass(s12, M, hk, src, dst, zden):
    mrep = jnp.broadcast_to(M[:, None], (HEADS, _L)).reshape(-1)
    den_parts, exbuf = _sc_den(src, dst, s12.reshape(-1), mrep, zden)
    den = den_parts[0] + den_parts[1]
    alpha = _sc_alpha(dst, exbuf, den[:, :HEADS].reshape(-1))
    hk2 = jnp.concatenate([hk[:, :UNITS // 2], hk[:, UNITS // 2:]], axis=0)
    return _sc_agg(src, dst, alpha, hk2)


def kernel(node_attributes, edge_indices, W_att, b_att, a_att, W_mlp, b_mlp,
           gamma, beta, Wi, Wh, b_lstm, Wp1, bp1, Wp2, bp2):
    x = node_attributes
    dst = edge_indices[:, 0].astype(jnp.int32)
    src = edge_indices[:, 1].astype(jnp.int32)

    a1 = np.zeros((HEADS * U, HEADS), np.float32)
    a2 = np.zeros((HEADS * U, HEADS), np.float32)
    mask1 = np.zeros((HEADS, U, HEADS), np.float32)
    mask2 = np.zeros((HEADS, U, HEADS), np.float32)
    for hd in range(HEADS):
        mask1[hd, :, hd] = 1.0
        mask2[hd, :, hd] = 1.0
    mask1 = jnp.asarray(mask1.reshape(HEADS * U, HEADS))
    mask2 = jnp.asarray(mask2.reshape(HEADS * U, HEADS))

    zden = jnp.zeros((_NP, _DW), jnp.float32)
    agg = None
    h = x
    upd = None
    for l in range(DEPTH):
        wl = jnp.transpose(W_att[l], (1, 0, 2)).reshape(D, HEADS * U)
        bl = b_att[l].reshape(1, HEADS * U)
        A1 = mask1 * a_att[l, :, :U].reshape(HEADS * U, 1)
        A2 = mask2 * a_att[l, :, U:2 * U].reshape(HEADS * U, 1)
        a12 = jnp.concatenate([A1, A2], axis=1)
        c12 = jnp.concatenate([jnp.zeros((HEADS,), jnp.float32), a_att[l, :, 2 * U]]).reshape(1, 2 * HEADS)
        gb = jnp.stack([gamma[l], beta[l]], axis=0)
        hk, s12, upd, h = _layer_tc(h if l == 0 else upd, agg, wl, bl, a12, c12,
                                    W_mlp[l], b_mlp[l].reshape(1, UNITS), gb)
        Mh = jnp.max(s12[:, :HEADS], axis=0) + jnp.max(s12[:, HEADS:], axis=0)
        M = jnp.maximum(Mh, 0.2 * Mh)
        agg = _edge_pass(s12, M, hk, src, dst, zden)

    p = _set2set_tc(upd, agg, Wi, Wh, b_lstm, Wp1, bp1, Wp2, bp2)
    return (p, jnp.stack([p, p], axis=1))


# speedup vs baseline: 23.9373x; 1.0440x over previous
import functools

import jax
import jax.numpy as jnp
import numpy as np
from jax import lax
from jax.experimental import pallas as pl
from jax.experimental.pallas import tpu as pltpu
from jax.experimental.pallas import tpu_sc as plsc

N = 10000
E = 320000
D = 128
UNITS = 128
HEADS = 4
U = UNITS // HEADS
DEPTH = 3
T = 3

_NC = 2
_NS = 16
_L = 16
_NW = _NC * _NS
_EP = 327680
_EW = _EP // _NW
_NP = 10240
_BLK = 128
_DW = 16
_NBLK = _EW // _BLK


def _sc_mesh(nc=_NC):
    return plsc.VectorSubcoreMesh(core_axis_name="c", subcore_axis_name="s",
                                  num_cores=nc, num_subcores=_NS)

_NB = 1000


def _layer_tc_body(h_ref, agg0_ref, agg1_ref, wl_ref, bl_ref, a12_ref, c12_ref,
                   wm_ref, bm_ref, gb_ref, hk_ref, s12_ref, upd_ref, h_out_ref):
    h = h_ref[...]
    if agg0_ref is not None:
        att = jnp.concatenate([agg0_ref[...], agg1_ref[...]], axis=1)
        h = h + jnp.maximum(att, 0.0)
    h_out_ref[...] = h
    hk = jnp.dot(h, wl_ref[...], preferred_element_type=jnp.float32) + bl_ref[...]
    hk_ref[...] = hk
    s12_ref[...] = jnp.dot(hk, a12_ref[...], preferred_element_type=jnp.float32) + c12_ref[...]
    upd = jnp.maximum(jnp.dot(h, wm_ref[...], preferred_element_type=jnp.float32) + bm_ref[...], 0.0)
    upd_ref[...] = upd * gb_ref[0:1, :] + gb_ref[1:2, :]


def _layer_tc(h_or_upd, agg, wl, bl, a12, c12, wm, bm, gb):
    first = agg is None

    def body_first(h_ref, wl_ref, bl_ref, a12_ref, c12_ref, wm_ref, bm_ref, gb_ref,
                   hk_ref, s12_ref, upd_ref, h_out_ref):
        _layer_tc_body(h_ref, None, None, wl_ref, bl_ref, a12_ref, c12_ref,
                       wm_ref, bm_ref, gb_ref, hk_ref, s12_ref, upd_ref, h_out_ref)

    row = lambda i: (i, 0)
    full = lambda i: (0, 0)
    in_specs = [pl.BlockSpec((_NB, UNITS), row)]
    args = [h_or_upd]
    if not first:
        in_specs += [pl.BlockSpec((_NB, UNITS // 2), row), pl.BlockSpec((_NB, UNITS // 2), row)]
        args += [agg[0], agg[1]]
    in_specs += [
        pl.BlockSpec((UNITS, UNITS), full),
        pl.BlockSpec((1, UNITS), full),
        pl.BlockSpec((UNITS, 2 * HEADS), full),
        pl.BlockSpec((1, 2 * HEADS), full),
        pl.BlockSpec((UNITS, UNITS), full),
        pl.BlockSpec((1, UNITS), full),
        pl.BlockSpec((2, UNITS), full),
    ]
    args += [wl, bl, a12, c12, wm, bm, gb]
    out_specs = [
        pl.BlockSpec((_NB, UNITS), row),
        pl.BlockSpec((_NB, 2 * HEADS), row),
        pl.BlockSpec((_NB, UNITS), row),
        pl.BlockSpec((_NB, UNITS), row),
    ]
    out_shape = [
        jax.ShapeDtypeStruct((N, UNITS), jnp.float32),
        jax.ShapeDtypeStruct((N, 2 * HEADS), jnp.float32),
        jax.ShapeDtypeStruct((N, UNITS), jnp.float32),
        jax.ShapeDtypeStruct((N, UNITS), jnp.float32),
    ]
    fn = body_first if first else _layer_tc_body
    return pl.pallas_call(
        fn, grid=(N // _NB,), in_specs=in_specs, out_specs=out_specs,
        out_shape=out_shape)(*args)


def _set2set_body(upd_ref, agg0_ref, agg1_ref, wi_ref, wh_ref, bl_ref,
                  wp1_ref, bp1_ref, wp2_ref, bp2_ref, out_ref):
    att = jnp.concatenate([agg0_ref[...], agg1_ref[...]], axis=1)
    h = upd_ref[...] + jnp.maximum(att, 0.0)
    C = UNITS
    q_star = jnp.zeros((1, 2 * C), jnp.float32)
    cs = jnp.zeros((1, C), jnp.float32)
    hs = jnp.zeros((1, C), jnp.float32)
    for _ in range(T):
        z = (jnp.dot(q_star, wi_ref[...], preferred_element_type=jnp.float32)
             + jnp.dot(hs, wh_ref[...], preferred_element_type=jnp.float32)
             + bl_ref[...])
        i = z[:, 0:C]
        f = z[:, C:2 * C]
        g = z[:, 2 * C:3 * C]
        o = z[:, 3 * C:4 * C]
        cs = jax.nn.sigmoid(f) * cs + jax.nn.sigmoid(i) * jnp.tanh(g)
        hs = jax.nn.sigmoid(o) * jnp.tanh(cs)
        q = hs
        e = jnp.sum(h * q, axis=1, keepdims=True)
        m = jnp.max(e)
        ea = jnp.exp(e - m)
        alpha = ea / jnp.sum(ea)
        r = jnp.sum(alpha * h, axis=0, keepdims=True)
        q_star = jnp.concatenate([q, r], axis=1)
    p = jnp.maximum(jnp.dot(q_star, wp1_ref[...], preferred_element_type=jnp.float32) + bp1_ref[...], 0.0)
    p = jnp.maximum(jnp.dot(p, wp2_ref[...], preferred_element_type=jnp.float32) + bp2_ref[...], 0.0)
    out_ref[...] = p


def _set2set_tc(upd, agg, Wi, Wh, b_lstm, Wp1, bp1, Wp2, bp2):
    C = UNITS
    zero2 = lambda i: (0, 0)
    in_specs = [
        pl.BlockSpec((N, C), zero2),
        pl.BlockSpec((N, C // 2), zero2),
        pl.BlockSpec((N, C // 2), zero2),
        pl.BlockSpec((2 * C, 4 * C), zero2),
        pl.BlockSpec((C, 4 * C), zero2),
        pl.BlockSpec((1, 4 * C), zero2),
        pl.BlockSpec((2 * C, C), zero2),
        pl.BlockSpec((1, C), zero2),
        pl.BlockSpec((C, C // 2), zero2),
        pl.BlockSpec((1, C // 2), zero2),
    ]
    return pl.pallas_call(
        _set2set_body,
        grid=(1,),
        in_specs=in_specs,
        out_specs=pl.BlockSpec((1, C // 2), zero2),
        out_shape=jax.ShapeDtypeStruct((1, UNITS // 2), jnp.float32),
    )(upd, agg[0], agg[1], Wi, Wh, b_lstm.reshape(1, -1),
      Wp1, bp1.reshape(1, -1), Wp2, bp2.reshape(1, -1))


def _sc_den(src, dst, s12flat, mrep, zden):
    @functools.partial(
        pl.kernel,
        out_type=(jax.ShapeDtypeStruct((_NC, _NP, _DW), jnp.float32),
                  jax.ShapeDtypeStruct((_EP * HEADS,), jnp.float32)),
        mesh=_sc_mesh(),
        compiler_params=pltpu.CompilerParams(needs_layout_passes=False, use_tc_tiling_on_sc=False),
        scratch_types=[
            pltpu.VMEM((_NP * 2 * HEADS,), jnp.float32),
            pltpu.VMEM((_BLK,), jnp.int32),
            pltpu.VMEM((_BLK,), jnp.int32),
            pltpu.VMEM((_BLK, _DW), jnp.float32),
            pltpu.VMEM((_BLK * HEADS,), jnp.float32),
            pltpu.VMEM((HEADS * _L,), jnp.float32),
            pltpu.VMEM_SHARED((_NP, _DW), jnp.float32),
        ],
    )
    def k(src_h, dst_h, s12_h, m_h, zden_h, den_out, ex_out,
          s12_v, srcb, dstb, exb, exbf, m_v, den_sh):
        c = lax.axis_index("c")
        s = lax.axis_index("s")
        base = c * (_EP // _NC) + s * _EW
        rows = _NP // _NS
        pltpu.sync_copy(zden_h.at[pl.ds(s * rows, rows)],
                        den_sh.at[pl.ds(s * rows, rows)])
        pltpu.sync_copy(s12_h, s12_v)
        pltpu.sync_copy(m_h, m_v)
        zv = jnp.zeros((_L,), jnp.float32)

        def zexr(e, carry):
            exb[e, pl.ds(0, _L)] = zv
            return carry

        lax.fori_loop(0, _BLK, zexr, 0)
        plsc.subcore_barrier()
        iota = lax.iota(jnp.int32, _L)

        def blk(j, carry):
            off = base + j * _BLK
            pltpu.sync_copy(src_h.at[pl.ds(off, _BLK)], srcb)
            pltpu.sync_copy(dst_h.at[pl.ds(off, _BLK)], dstb)

            def grp(i, carry2):
                srcv = srcb[pl.ds(i * _L, _L)]
                dstv = dstb[pl.ds(i * _L, _L)]
                si = srcv * (2 * HEADS)
                di = dstv * (2 * HEADS) + HEADS
                row = i * _L + iota
                for hd in range(HEADS):
                    s1 = plsc.load_gather(s12_v, [si + hd])
                    s2 = plsc.load_gather(s12_v, [di + hd])
                    z = s1 + s2
                    lg = jnp.maximum(z, 0.2 * z)
                    exv = jnp.exp(lg - m_v[pl.ds(hd * _L, _L)])
                    plsc.store_scatter(exb, [row, jnp.full((_L,), hd, jnp.int32)], exv)
                    plsc.store_scatter(exbf, [row * HEADS + hd], exv)
                return carry2

            lax.fori_loop(0, _BLK // _L, grp, 0)
            pltpu.sync_copy(exbf, ex_out.at[pl.ds(off * HEADS, _BLK * HEADS)])
            pltpu.sync_copy(exb, den_sh.at[dstb], add=True)
            return carry

        lax.fori_loop(0, _NBLK, blk, 0)
        plsc.subcore_barrier()
        pltpu.sync_copy(den_sh.at[pl.ds(s * rows, rows)],
                        den_out.at[c, pl.ds(s * rows, rows)])

    return k(src, dst, s12flat, mrep, zden)


def _sc_alpha(dst, exbuf, den):
    @functools.partial(
        pl.kernel,
        out_type=jax.ShapeDtypeStruct((_EP * HEADS,), jnp.float32),
        mesh=_sc_mesh(),
        compiler_params=pltpu.CompilerParams(needs_layout_passes=False),
        scratch_types=[
            pltpu.VMEM((_BLK,), jnp.int32),
            pltpu.VMEM((_BLK * HEADS,), jnp.float32),
            pltpu.VMEM((_NP * HEADS,), jnp.float32),
            pltpu.VMEM((_BLK * HEADS,), jnp.float32),
            pltpu.SemaphoreType.DMA,
        ],
    )
    def k(dst_h, ex_h, den_h, al_out, dstb, exb, den_v, ab, sem):
        c = lax.axis_index("c")
        s = lax.axis_index("s")
        base = c * (_EP // _NC) + s * _EW
        iota = lax.iota(jnp.int32, _L)
        pltpu.sync_copy(den_h, den_v)

        def blk(j, carry):
            off = base + j * _BLK
            pltpu.sync_copy(dst_h.at[pl.ds(off, _BLK)], dstb)
            pltpu.sync_copy(ex_h.at[pl.ds(off * HEADS, _BLK * HEADS)], exb)

            def grp(i, carry2):
                dstv = dstb[pl.ds(i * _L, _L)]
                row = i * _L + iota
                for hd in range(HEADS):
                    exv = plsc.load_gather(exb, [row * HEADS + hd])
                    denv = plsc.load_gather(den_v, [dstv * HEADS + hd])
                    plsc.store_scatter(ab, [row * HEADS + hd], exv / (denv + 1e-9))
                return carry2

            lax.fori_loop(0, _BLK // _L, grp, 0)
            pltpu.sync_copy(ab, al_out.at[pl.ds(off * HEADS, _BLK * HEADS)])
            return carry

        lax.fori_loop(0, _NBLK, blk, 0)

    return k(dst, exbuf, den)


def _sc_agg(src, dst, alpha, hk2):
    HALF = UNITS // 2

    @functools.partial(
        pl.kernel,
        out_type=jax.ShapeDtypeStruct((_NC, _NP, HALF), jnp.float32),
        mesh=_sc_mesh(),
        compiler_params=pltpu.CompilerParams(needs_layout_passes=False, use_tc_tiling_on_sc=False),
        scratch_types=[
            pltpu.VMEM((_BLK,), jnp.int32),
            pltpu.VMEM((_BLK,), jnp.int32),
            pltpu.VMEM((_BLK * HEADS,), jnp.float32),
            pltpu.VMEM((_BLK, HALF), jnp.float32),
            pltpu.VMEM_SHARED((_NP, HALF), jnp.float32),
            pltpu.SemaphoreType.DMA,
        ],
    )
    def k(src_h, dst_h, al_h, hk_h, agg_out,
          srcb, dstb, ab, rows_v, agg_sh, sem):
        c = lax.axis_index("c")
        s = lax.axis_index("s")
        base = s * (_EP // _NS)
        rows = _NP // _NS
        nblk = (_EP // _NS) // _BLK
        zv = jnp.zeros((_L,), jnp.float32)
        roff = c * N

        def zrow(e, carry):
            for kk in range(HALF // _L):
                rows_v[e, pl.ds(kk * _L, _L)] = zv
            return carry

        lax.fori_loop(0, _BLK, zrow, 0)
        for i in range(rows // _BLK):
            pltpu.sync_copy(rows_v, agg_sh.at[pl.ds(s * rows + i * _BLK, _BLK)])
        plsc.subcore_barrier()

        def blk(j, carry):
            off = base + j * _BLK
            pltpu.sync_copy(src_h.at[pl.ds(off, _BLK)], srcb)
            pltpu.sync_copy(dst_h.at[pl.ds(off, _BLK)], dstb)
            pltpu.sync_copy(al_h.at[pl.ds(off * HEADS, _BLK * HEADS)], ab)

            def shft(i, carry0):
                sl = pl.ds(i * _L, _L)
                srcb[sl] = srcb[sl] + roff
                return carry0

            lax.fori_loop(0, _BLK // _L, shft, 0)
            pltpu.async_copy(hk_h.at[srcb], rows_v, sem).wait()

            def edge(e, carry3):
                for hd in range(HEADS // _NC):
                    av = plsc.load_gather(
                        ab, [jnp.full((_L,), e * HEADS + hd, jnp.int32) + 2 * c])
                    for kk in range(U // _L):
                        c0 = hd * U + kk * _L
                        rows_v[e, pl.ds(c0, _L)] = rows_v[e, pl.ds(c0, _L)] * av
                return carry3

            lax.fori_loop(0, _BLK, edge, 0)
            pltpu.sync_copy(rows_v, agg_sh.at[dstb], add=True)
            return carry

        lax.fori_loop(0, nblk, blk, 0)
        plsc.subcore_barrier()
        pltpu.sync_copy(agg_sh.at[pl.ds(s * rows, rows)],
                        agg_out.at[c, pl.ds(s * rows, rows)])

    return k(src, dst, alpha, hk2)


def _edge_pass(s12, M, hk, src, dst, zden):
    mrep = jnp.broadcast_to(M[:, None], (HEADS, _L)).reshape(-1)
    s12p = jnp.concatenate(
        [s12.reshape(-1), jnp.full(((_NP - N) * 2 * HEADS,), -1e30, jnp.float32)])
    den_parts, exbuf = _sc_den(src, dst, s12p, mrep, zden)
    den = den_parts[0] + den_parts[1]
    alpha = _sc_alpha(dst, exbuf, den[:, :HEADS].reshape(-1))
    hk2 = jnp.concatenate([hk[:, :UNITS // 2], hk[:, UNITS // 2:]], axis=0)
    return _sc_agg(src, dst, alpha, hk2)


def kernel(node_attributes, edge_indices, W_att, b_att, a_att, W_mlp, b_mlp,
           gamma, beta, Wi, Wh, b_lstm, Wp1, bp1, Wp2, bp2):
    x = node_attributes
    dst = jnp.concatenate([edge_indices[:, 0].astype(jnp.int32),
                           jnp.full((_EP - E,), N, jnp.int32)])
    src = jnp.concatenate([edge_indices[:, 1].astype(jnp.int32),
                           jnp.zeros((_EP - E,), jnp.int32)])

    a1 = np.zeros((HEADS * U, HEADS), np.float32)
    a2 = np.zeros((HEADS * U, HEADS), np.float32)
    mask1 = np.zeros((HEADS, U, HEADS), np.float32)
    mask2 = np.zeros((HEADS, U, HEADS), np.float32)
    for hd in range(HEADS):
        mask1[hd, :, hd] = 1.0
        mask2[hd, :, hd] = 1.0
    mask1 = jnp.asarray(mask1.reshape(HEADS * U, HEADS))
    mask2 = jnp.asarray(mask2.reshape(HEADS * U, HEADS))

    zden = jnp.zeros((_NP, _DW), jnp.float32)
    agg = None
    h = x
    upd = None
    for l in range(DEPTH):
        wl = jnp.transpose(W_att[l], (1, 0, 2)).reshape(D, HEADS * U)
        bl = b_att[l].reshape(1, HEADS * U)
        A1 = mask1 * a_att[l, :, :U].reshape(HEADS * U, 1)
        A2 = mask2 * a_att[l, :, U:2 * U].reshape(HEADS * U, 1)
        a12 = jnp.concatenate([A1, A2], axis=1)
        c12 = jnp.concatenate([jnp.zeros((HEADS,), jnp.float32), a_att[l, :, 2 * U]]).reshape(1, 2 * HEADS)
        gb = jnp.stack([gamma[l], beta[l]], axis=0)
        hk, s12, upd, h = _layer_tc(h if l == 0 else upd, agg, wl, bl, a12, c12,
                                    W_mlp[l], b_mlp[l].reshape(1, UNITS), gb)
        Mh = jnp.max(s12[:, :HEADS], axis=0) + jnp.max(s12[:, HEADS:], axis=0)
        M = jnp.maximum(Mh, 0.2 * Mh)
        agg = _edge_pass(s12, M, hk, src, dst, zden)

    p = _set2set_tc(upd, agg, Wi, Wh, b_lstm, Wp1, bp1, Wp2, bp2)
    return (p, jnp.stack([p, p], axis=1))
